# Initial kernel scaffold; baseline (speedup 1.0000x reference)
import jax
import jax.numpy as jnp
from jax.experimental import pallas as pl


def kernel(q, K, V, salience, topk):
    raise NotImplementedError("write your pallas kernel here")


# speedup vs baseline: 4.5412x; 4.5412x over previous
import functools
import math

import jax
import jax.numpy as jnp
from jax.experimental import pallas as pl


def _body(q_ref, k_ref, v_ref, sal_ref, rv_ref, w_ref, *, kk, scale):
    scores = jax.lax.dot_general(
        q_ref[...], k_ref[...],
        dimension_numbers=(((1,), (1,)), ((), ())),
        preferred_element_type=jnp.float32,
    ) * scale + sal_ref[...]

    tile, s = scores.shape
    col = jax.lax.broadcasted_iota(jnp.int32, (tile, s), 1)

    cur = scores
    vals = []
    idxs = []
    for _ in range(kk):
        mj = jnp.max(cur, axis=1, keepdims=True)
        ismax = cur == mj
        ij = jnp.min(jnp.where(ismax, col, s), axis=1, keepdims=True)
        vals.append(mj)
        idxs.append(ij)
        cur = jnp.where(col == ij, -jnp.inf, cur)

    m = vals[0]
    exps = [jnp.exp(v - m) for v in vals]
    denom = functools.reduce(jnp.add, exps)
    inv = 1.0 / denom

    w_ref[...] = jnp.concatenate([e * inv for e in exps], axis=1)

    w_dense = jnp.zeros((tile, s), dtype=jnp.float32)
    for ij, e in zip(idxs, exps):
        w_dense = w_dense + jnp.where(col == ij, e * inv, 0.0)

    rv_ref[...] = jax.lax.dot_general(
        w_dense, v_ref[...],
        dimension_numbers=(((1,), (0,)), ((), ())),
        preferred_element_type=jnp.float32,
        precision=jax.lax.Precision.HIGHEST,
    )


def _run(q2, K, V, sal2, kk, tile, interpret=False):
    n, d = q2.shape
    s = K.shape[0]
    grid = (n // tile,)
    body = functools.partial(_body, kk=kk, scale=1.0 / math.sqrt(d))
    rv, w = pl.pallas_call(
        body,
        grid=grid,
        in_specs=[
            pl.BlockSpec((tile, d), lambda i: (i, 0)),
            pl.BlockSpec((s, d), lambda i: (0, 0)),
            pl.BlockSpec((s, d), lambda i: (0, 0)),
            pl.BlockSpec((1, s), lambda i: (0, 0)),
        ],
        out_specs=[
            pl.BlockSpec((tile, d), lambda i: (i, 0)),
            pl.BlockSpec((tile, kk), lambda i: (i, 0)),
        ],
        out_shape=[
            jax.ShapeDtypeStruct((n, d), jnp.float32),
            jax.ShapeDtypeStruct((n, kk), jnp.float32),
        ],
        interpret=interpret,
    )(q2, K, V, sal2)
    return rv, w


def kernel(q, K, V, salience, topk):
    b, t, d = q.shape
    s = K.shape[0]
    kk = min(8, s)
    n = b * t
    q2 = q.reshape(n, d)
    sal2 = salience.reshape(1, s)
    tile = 256
    rv, w = _run(q2, K, V, sal2, kk, tile)
    return rv.reshape(b, t, d), w.reshape(b, t, kk)


# speedup vs baseline: 9.4432x; 2.0795x over previous
import functools
import math

import jax
import jax.numpy as jnp
from jax.experimental import pallas as pl


def _body(q_ref, k_ref, v_ref, sal_ref, rv_ref, w_ref, *, kk, scale):
    scores = jax.lax.dot_general(
        q_ref[...], k_ref[...],
        dimension_numbers=(((1,), (1,)), ((), ())),
        preferred_element_type=jnp.float32,
    ) * scale + sal_ref[...]

    tile, s = scores.shape

    cur = scores
    vals = []
    masks = []
    for _ in range(kk):
        mj = jnp.max(cur, axis=1, keepdims=True)
        ismax = cur == mj
        vals.append(mj)
        masks.append(ismax)
        cur = jnp.where(ismax, -jnp.inf, cur)

    m = vals[0]
    exps = [jnp.exp(v - m) for v in vals]
    denom = functools.reduce(jnp.add, exps)
    inv = 1.0 / denom

    w_ref[...] = jnp.concatenate([e * inv for e in exps], axis=1)

    w_dense = jnp.zeros((tile, s), dtype=jnp.float32)
    for msk, e in zip(masks, exps):
        w_dense = w_dense + jnp.where(msk, e * inv, 0.0)

    rv_ref[...] = jax.lax.dot_general(
        w_dense, v_ref[...],
        dimension_numbers=(((1,), (0,)), ((), ())),
        preferred_element_type=jnp.float32,
    )


def _run(q2, K, V, sal2, kk, tile, interpret=False):
    n, d = q2.shape
    s = K.shape[0]
    grid = (n // tile,)
    body = functools.partial(_body, kk=kk, scale=1.0 / math.sqrt(d))
    rv, w = pl.pallas_call(
        body,
        grid=grid,
        in_specs=[
            pl.BlockSpec((tile, d), lambda i: (i, 0)),
            pl.BlockSpec((s, d), lambda i: (0, 0)),
            pl.BlockSpec((s, d), lambda i: (0, 0)),
            pl.BlockSpec((1, s), lambda i: (0, 0)),
        ],
        out_specs=[
            pl.BlockSpec((tile, d), lambda i: (i, 0)),
            pl.BlockSpec((tile, kk), lambda i: (i, 0)),
        ],
        out_shape=[
            jax.ShapeDtypeStruct((n, d), jnp.float32),
            jax.ShapeDtypeStruct((n, kk), jnp.float32),
        ],
        interpret=interpret,
    )(q2, K, V, sal2)
    return rv, w


def kernel(q, K, V, salience, topk):
    b, t, d = q.shape
    s = K.shape[0]
    kk = min(8, s)
    n = b * t
    q2 = q.reshape(n, d)
    sal2 = salience.reshape(1, s)
    tile = 256
    rv, w = _run(q2, K, V, sal2, kk, tile)
    return rv.reshape(b, t, d), w.reshape(b, t, kk)


# speedup vs baseline: 14.0010x; 1.4827x over previous
import functools
import math

import jax
import jax.numpy as jnp
from jax.experimental import pallas as pl


def _body(q_ref, k_ref, v_ref, sal_ref, rv_ref, w_ref, *, kk, scale):
    scores = jax.lax.dot_general(
        q_ref[...], k_ref[...],
        dimension_numbers=(((1,), (1,)), ((), ())),
        preferred_element_type=jnp.float32,
    ) * scale + sal_ref[...]

    tile, s = scores.shape

    cur = scores
    m = None
    exps = []
    w_e = None
    for _ in range(kk):
        mj = jnp.max(cur, axis=1, keepdims=True)
        ismax = cur == mj
        e = jnp.ones_like(mj) if m is None else jnp.exp(mj - m)
        if m is None:
            m = mj
        exps.append(e)
        upd = jnp.where(ismax, e, 0.0)
        w_e = upd if w_e is None else w_e + upd
        cur = jnp.where(ismax, -jnp.inf, cur)

    denom = functools.reduce(jnp.add, exps)
    inv = 1.0 / denom

    w_ref[...] = jnp.concatenate(exps, axis=1) * inv
    w_dense = w_e * inv

    rv_ref[...] = jax.lax.dot_general(
        w_dense, v_ref[...],
        dimension_numbers=(((1,), (0,)), ((), ())),
        preferred_element_type=jnp.float32,
    )


def _run(q2, K, V, sal2, kk, tile, interpret=False):
    n, d = q2.shape
    s = K.shape[0]
    grid = (n // tile,)
    body = functools.partial(_body, kk=kk, scale=1.0 / math.sqrt(d))
    rv, w = pl.pallas_call(
        body,
        grid=grid,
        in_specs=[
            pl.BlockSpec((tile, d), lambda i: (i, 0)),
            pl.BlockSpec((s, d), lambda i: (0, 0)),
            pl.BlockSpec((s, d), lambda i: (0, 0)),
            pl.BlockSpec((1, s), lambda i: (0, 0)),
        ],
        out_specs=[
            pl.BlockSpec((tile, d), lambda i: (i, 0)),
            pl.BlockSpec((tile, kk), lambda i: (i, 0)),
        ],
        out_shape=[
            jax.ShapeDtypeStruct((n, d), jnp.float32),
            jax.ShapeDtypeStruct((n, kk), jnp.float32),
        ],
        interpret=interpret,
    )(q2, K, V, sal2)
    return rv, w


def kernel(q, K, V, salience, topk):
    b, t, d = q.shape
    s = K.shape[0]
    kk = min(8, s)
    n = b * t
    q2 = q.reshape(n, d)
    sal2 = salience.reshape(1, s)
    tile = 512
    rv, w = _run(q2, K, V, sal2, kk, tile)
    return rv.reshape(b, t, d), w.reshape(b, t, kk)


# speedup vs baseline: 16.7697x; 1.1977x over previous
import functools
import math

import jax
import jax.numpy as jnp
from jax.experimental import pallas as pl


def _body(q_ref, k_ref, v_ref, sal_ref, rv_ref, w_ref, *, kk, scale):
    scores = jax.lax.dot_general(
        q_ref[...], k_ref[...],
        dimension_numbers=(((1,), (1,)), ((), ())),
        preferred_element_type=jnp.float32,
    ) * scale + sal_ref[...]

    tile, s = scores.shape

    cur = scores
    m = None
    exps = []
    w_e = None
    for _ in range(kk):
        mj = jnp.max(cur, axis=1, keepdims=True)
        ismax = cur == mj
        e = jnp.ones_like(mj) if m is None else jnp.exp(mj - m)
        if m is None:
            m = mj
        exps.append(e)
        upd = jnp.where(ismax, e, 0.0)
        w_e = upd if w_e is None else w_e + upd
        cur = jnp.where(ismax, -jnp.inf, cur)

    denom = functools.reduce(jnp.add, exps)
    inv = 1.0 / denom

    w_ref[...] = jnp.concatenate(exps, axis=1) * inv
    w_dense = w_e * inv

    rv_ref[...] = jax.lax.dot_general(
        w_dense, v_ref[...],
        dimension_numbers=(((1,), (0,)), ((), ())),
        preferred_element_type=jnp.float32,
    )


def _run(q2, K, V, sal2, kk, tile, interpret=False):
    n, d = q2.shape
    s = K.shape[0]
    grid = (n // tile,)
    body = functools.partial(_body, kk=kk, scale=1.0 / math.sqrt(d))
    rv, w = pl.pallas_call(
        body,
        grid=grid,
        in_specs=[
            pl.BlockSpec((tile, d), lambda i: (i, 0)),
            pl.BlockSpec((s, d), lambda i: (0, 0)),
            pl.BlockSpec((s, d), lambda i: (0, 0)),
            pl.BlockSpec((1, s), lambda i: (0, 0)),
        ],
        out_specs=[
            pl.BlockSpec((tile, d), lambda i: (i, 0)),
            pl.BlockSpec((tile, kk), lambda i: (i, 0)),
        ],
        out_shape=[
            jax.ShapeDtypeStruct((n, d), jnp.float32),
            jax.ShapeDtypeStruct((n, kk), jnp.float32),
        ],
        interpret=interpret,
    )(q2, K, V, sal2)
    return rv, w


def kernel(q, K, V, salience, topk):
    b, t, d = q.shape
    s = K.shape[0]
    kk = min(8, s)
    n = b * t
    q2 = q.reshape(n, d)
    sal2 = salience.reshape(1, s)
    tile = 1024
    rv, w = _run(q2, K, V, sal2, kk, tile)
    return rv.reshape(b, t, d), w.reshape(b, t, kk)


# speedup vs baseline: 17.3992x; 1.0375x over previous
import functools
import math

import jax
import jax.numpy as jnp
from jax.experimental import pallas as pl


def _body(q_ref, k_ref, v_ref, sal_ref, rv_ref, w_ref, *, kk, scale):
    scores = jax.lax.dot_general(
        q_ref[...], k_ref[...],
        dimension_numbers=(((1,), (1,)), ((), ())),
        preferred_element_type=jnp.float32,
    ) * scale + sal_ref[...]

    tile, s = scores.shape

    cur = scores
    m = None
    exps = []
    w_e = None
    for _ in range(kk):
        mj = jnp.max(cur, axis=1, keepdims=True)
        ismax = cur == mj
        e = jnp.ones_like(mj) if m is None else jnp.exp(mj - m)
        if m is None:
            m = mj
        exps.append(e)
        upd = jnp.where(ismax, e, 0.0)
        w_e = upd if w_e is None else w_e + upd
        cur = jnp.where(ismax, -jnp.inf, cur)

    denom = functools.reduce(jnp.add, exps)
    inv = 1.0 / denom

    w_ref[...] = jnp.concatenate(exps, axis=1) * inv
    w_dense = w_e * inv

    rv_ref[...] = jax.lax.dot_general(
        w_dense, v_ref[...],
        dimension_numbers=(((1,), (0,)), ((), ())),
        preferred_element_type=jnp.float32,
    )


def _run(q2, K, V, sal2, kk, tile, interpret=False):
    n, d = q2.shape
    s = K.shape[0]
    grid = (n // tile,)
    body = functools.partial(_body, kk=kk, scale=1.0 / math.sqrt(d))
    rv, w = pl.pallas_call(
        body,
        grid=grid,
        in_specs=[
            pl.BlockSpec((tile, d), lambda i: (i, 0)),
            pl.BlockSpec((s, d), lambda i: (0, 0)),
            pl.BlockSpec((s, d), lambda i: (0, 0)),
            pl.BlockSpec((1, s), lambda i: (0, 0)),
        ],
        out_specs=[
            pl.BlockSpec((tile, d), lambda i: (i, 0)),
            pl.BlockSpec((tile, kk), lambda i: (i, 0)),
        ],
        out_shape=[
            jax.ShapeDtypeStruct((n, d), jnp.float32),
            jax.ShapeDtypeStruct((n, kk), jnp.float32),
        ],
        interpret=interpret,
    )(q2, K, V, sal2)
    return rv, w


def kernel(q, K, V, salience, topk):
    b, t, d = q.shape
    s = K.shape[0]
    kk = min(8, s)
    n = b * t
    q2 = q.reshape(n, d)
    sal2 = salience.reshape(1, s)
    tile = 2048
    rv, w = _run(q2, K, V, sal2, kk, tile)
    return rv.reshape(b, t, d), w.reshape(b, t, kk)
